# Initial kernel scaffold; baseline (speedup 1.0000x reference)
import functools

import jax
import jax.numpy as jnp
from jax import lax
from jax.experimental import pallas as pl
from jax.experimental.pallas import tpu as pltpu
from jax.experimental.pallas import tpu_sc as plsc

N = 10000
E = 320000
D = 128
O = 2

NC = 2
NS = 16
NW = NC * NS
CB = 100
ER = E // CB
KJ = E // (NW * CB)
ZS = N // 10


def _sc_mesh():
    return plsc.VectorSubcoreMesh(core_axis_name="c", subcore_axis_name="s")


def _sc_degree(dst2, ones_u, zeros_n):
    @functools.partial(
        pl.kernel,
        out_type=jax.ShapeDtypeStruct((NC, N), jnp.float32),
        mesh=_sc_mesh(),
        scratch_types=[
            pltpu.VMEM((KJ, CB), jnp.int32),
            pltpu.VMEM((CB,), jnp.float32),
            pltpu.VMEM_SHARED((N,), jnp.float32),
        ],
    )
    def k(dst_hbm, ones_hbm, zeros_hbm, out_hbm, dst_v, ones_v, deg_sh):
        c = lax.axis_index("c")
        s = lax.axis_index("s")
        wid = s * NC + c

        @pl.when(s < 10)
        def _():
            pltpu.sync_copy(zeros_hbm.at[pl.ds(s * ZS, ZS)],
                            deg_sh.at[pl.ds(s * ZS, ZS)])

        pltpu.sync_copy(ones_hbm, ones_v)
        pltpu.sync_copy(dst_hbm.at[pl.ds(wid * KJ, KJ)], dst_v)
        plsc.subcore_barrier()

        def body(j, carry):
            pltpu.sync_copy(ones_v, deg_sh.at[dst_v.at[j]], add=True)
            return carry

        lax.fori_loop(0, KJ, body, 0)
        plsc.subcore_barrier()

        @pl.when(s < 10)
        def _():
            pltpu.sync_copy(deg_sh.at[pl.ds(s * ZS, ZS)],
                            out_hbm.at[c, pl.ds(s * ZS, ZS)])

    return k(dst2, ones_u, zeros_n)


def _sc_scatter_rows(y, src2, dst2, zeros_nd):
    @functools.partial(
        pl.kernel,
        out_type=jax.ShapeDtypeStruct((NC, N, D), jnp.float32),
        mesh=_sc_mesh(),
        scratch_types=[
            pltpu.VMEM((KJ, CB), jnp.int32),
            pltpu.VMEM((KJ, CB), jnp.int32),
            pltpu.VMEM((CB, D), jnp.float32),
            pltpu.VMEM_SHARED((N, D), jnp.float32),
        ],
    )
    def k(y_hbm, src_hbm, dst_hbm, zeros_hbm, out_hbm,
          src_v, dst_v, rows_v, acc_sh):
        c = lax.axis_index("c")
        s = lax.axis_index("s")
        wid = s * NC + c

        @pl.when(s < 10)
        def _():
            pltpu.sync_copy(zeros_hbm.at[pl.ds(s * ZS, ZS)],
                            acc_sh.at[pl.ds(s * ZS, ZS)])

        pltpu.sync_copy(src_hbm.at[pl.ds(wid * KJ, KJ)], src_v)
        pltpu.sync_copy(dst_hbm.at[pl.ds(wid * KJ, KJ)], dst_v)
        plsc.subcore_barrier()

        def body(j, carry):
            pltpu.sync_copy(y_hbm.at[src_v.at[j]], rows_v)
            pltpu.sync_copy(rows_v, acc_sh.at[dst_v.at[j]], add=True)
            return carry

        lax.fori_loop(0, KJ, body, 0)
        plsc.subcore_barrier()

        @pl.when(s < 10)
        def _():
            pltpu.sync_copy(acc_sh.at[pl.ds(s * ZS, ZS)],
                            out_hbm.at[c, pl.ds(s * ZS, ZS)])

    return k(y, src2, dst2, zeros_nd)


def _tc_stage1(deg_t, x, w1):
    def body(deg_ref, x_ref, w_ref, dis_ref, y_ref):
        deg = deg_ref[:, 0:1] + deg_ref[:, 1:2] + 1.0
        dis = lax.rsqrt(deg)
        dis_ref[...] = dis
        xw = jnp.dot(x_ref[...], w_ref[...],
                     preferred_element_type=jnp.float32,
                     precision=lax.Precision.HIGHEST)
        y_ref[...] = xw * dis

    return pl.pallas_call(
        body,
        out_shape=[jax.ShapeDtypeStruct((N, 1), jnp.float32),
                   jax.ShapeDtypeStruct((N, D), jnp.float32)],
    )(deg_t, x, w1)


def _tc_stage2(accp, y1, dis, w2, b1):
    def body(a_ref, y_ref, d_ref, w_ref, b_ref, y2_ref):
        acc = a_ref[0] + a_ref[1] + y_ref[...]
        h = jnp.maximum(d_ref[...] * acc + b_ref[...], 0.0)
        hw = jnp.dot(h, w_ref[...],
                     preferred_element_type=jnp.float32,
                     precision=lax.Precision.HIGHEST)
        y2_ref[...] = hw * d_ref[...]

    return pl.pallas_call(
        body,
        out_shape=jax.ShapeDtypeStruct((N, D), jnp.float32),
    )(accp, y1, dis, w2, b1)


def _tc_stage3(accp, y2, dis, w3, b2, b3):
    def body(a_ref, y_ref, d_ref, w_ref, b2_ref, b3_ref, o_ref):
        acc = a_ref[0] + a_ref[1] + y_ref[...]
        h = jnp.maximum(d_ref[...] * acc + b2_ref[...], 0.0)
        logits = jnp.dot(h, w_ref[...],
                         preferred_element_type=jnp.float32,
                         precision=lax.Precision.HIGHEST) + b3_ref[...]
        m = jnp.max(logits, axis=-1, keepdims=True)
        lse = m + jnp.log(jnp.sum(jnp.exp(logits - m), axis=-1, keepdims=True))
        o_ref[...] = logits - lse

    return pl.pallas_call(
        body,
        out_shape=jax.ShapeDtypeStruct((N, O), jnp.float32),
    )(accp, y2, dis, w3, b2, b3)


def kernel(x, edge_index, W1, b1, W2, b2, W3, b3):
    src2 = edge_index[0].reshape(ER, CB)
    dst2 = edge_index[1].reshape(ER, CB)
    ones_u = jnp.ones((CB,), jnp.float32)
    zeros_n = jnp.zeros((N,), jnp.float32)
    zeros_nd = jnp.zeros((N, D), jnp.float32)

    deg_parts = _sc_degree(dst2, ones_u, zeros_n)
    deg_t = deg_parts.T
    dis, y1 = _tc_stage1(deg_t, x, W1)
    acc1 = _sc_scatter_rows(y1, src2, dst2, zeros_nd)
    y2 = _tc_stage2(acc1, y1, dis, W2, b1)
    acc2 = _sc_scatter_rows(y2, src2, dst2, zeros_nd)
    return _tc_stage3(acc2, y2, dis, W3, b2, b3)


# speedup vs baseline: 18.8350x; 18.8350x over previous
import functools

import jax
import jax.numpy as jnp
from jax import lax
from jax.experimental import pallas as pl
from jax.experimental.pallas import tpu as pltpu
from jax.experimental.pallas import tpu_sc as plsc

N = 10000
E = 320000
D = 128
O = 2

NC = 2
NS = 16
NW = NC * NS
CB = 100
KJ = E // (NW * CB)
ZS = N // 10


def _sc_mesh():
    return plsc.VectorSubcoreMesh(core_axis_name="c", subcore_axis_name="s")


def _sc_degree(dst3, ones_u, zeros_n):
    @functools.partial(
        pl.kernel,
        out_type=jax.ShapeDtypeStruct((NC, N, D), jnp.float32),
        mesh=_sc_mesh(),
        scratch_types=[
            pltpu.VMEM((KJ, CB), jnp.int32),
            pltpu.VMEM((CB, D), jnp.float32),
            pltpu.VMEM_SHARED((N, D), jnp.float32),
        ],
    )
    def k(dst_hbm, ones_hbm, zeros_hbm, out_hbm, dst_v, ones_v, deg_sh):
        c = lax.axis_index("c")
        s = lax.axis_index("s")
        wid = s * NC + c

        @pl.when(s < 10)
        def _():
            pltpu.sync_copy(zeros_hbm.at[pl.ds(s * ZS, ZS)],
                            deg_sh.at[pl.ds(s * ZS, ZS)])

        pltpu.sync_copy(ones_hbm, ones_v)
        pltpu.sync_copy(dst_hbm.at[wid], dst_v)
        plsc.subcore_barrier()

        def body(j, carry):
            pltpu.sync_copy(ones_v, deg_sh.at[dst_v.at[j]], add=True)
            return carry

        lax.fori_loop(0, KJ, body, 0)
        plsc.subcore_barrier()

        @pl.when(s < 10)
        def _():
            pltpu.sync_copy(deg_sh.at[pl.ds(s * ZS, ZS)],
                            out_hbm.at[c, pl.ds(s * ZS, ZS)])

    return k(dst3, ones_u, zeros_n)


def _sc_scatter_rows(y, src3, dst3, zeros_nd):
    @functools.partial(
        pl.kernel,
        out_type=jax.ShapeDtypeStruct((NC, N, D), jnp.float32),
        mesh=_sc_mesh(),
        scratch_types=[
            pltpu.VMEM((KJ, CB), jnp.int32),
            pltpu.VMEM((KJ, CB), jnp.int32),
            pltpu.VMEM((CB, D), jnp.float32),
            pltpu.VMEM_SHARED((N, D), jnp.float32),
        ],
    )
    def k(y_hbm, src_hbm, dst_hbm, zeros_hbm, out_hbm,
          src_v, dst_v, rows_v, acc_sh):
        c = lax.axis_index("c")
        s = lax.axis_index("s")
        wid = s * NC + c

        @pl.when(s < 10)
        def _():
            pltpu.sync_copy(zeros_hbm.at[pl.ds(s * ZS, ZS)],
                            acc_sh.at[pl.ds(s * ZS, ZS)])

        pltpu.sync_copy(src_hbm.at[wid], src_v)
        pltpu.sync_copy(dst_hbm.at[wid], dst_v)
        plsc.subcore_barrier()

        def body(j, carry):
            pltpu.sync_copy(y_hbm.at[src_v.at[j]], rows_v)
            pltpu.sync_copy(rows_v, acc_sh.at[dst_v.at[j]], add=True)
            return carry

        lax.fori_loop(0, KJ, body, 0)
        plsc.subcore_barrier()

        @pl.when(s < 10)
        def _():
            pltpu.sync_copy(acc_sh.at[pl.ds(s * ZS, ZS)],
                            out_hbm.at[c, pl.ds(s * ZS, ZS)])

    return k(y, src3, dst3, zeros_nd)


def _tc_stage1(deg_t, x, w1):
    def body(deg_ref, x_ref, w_ref, dis_ref, y_ref):
        d0 = deg_ref[0]
        d1 = deg_ref[1]
        deg = d0[:, 0:1] + d1[:, 0:1] + 1.0
        dis = lax.rsqrt(deg)
        dis_ref[...] = dis
        xw = jnp.dot(x_ref[...], w_ref[...],
                     preferred_element_type=jnp.float32,
                     precision=lax.Precision.HIGHEST)
        y_ref[...] = xw * dis

    return pl.pallas_call(
        body,
        out_shape=[jax.ShapeDtypeStruct((N, 1), jnp.float32),
                   jax.ShapeDtypeStruct((N, D), jnp.float32)],
    )(deg_t, x, w1)


def _tc_stage2(accp, y1, dis, w2, b1):
    def body(a_ref, y_ref, d_ref, w_ref, b_ref, y2_ref):
        acc = a_ref[0] + a_ref[1] + y_ref[...]
        h = jnp.maximum(d_ref[...] * acc + b_ref[...], 0.0)
        hw = jnp.dot(h, w_ref[...],
                     preferred_element_type=jnp.float32,
                     precision=lax.Precision.HIGHEST)
        y2_ref[...] = hw * d_ref[...]

    return pl.pallas_call(
        body,
        out_shape=jax.ShapeDtypeStruct((N, D), jnp.float32),
    )(accp, y1, dis, w2, b1)


def _tc_stage3(accp, y2, dis, w3, b2, b3):
    def body(a_ref, y_ref, d_ref, w_ref, b2_ref, b3_ref, o_ref):
        acc = a_ref[0] + a_ref[1] + y_ref[...]
        h = jnp.maximum(d_ref[...] * acc + b2_ref[...], 0.0)
        logits = jnp.dot(h, w_ref[...],
                         preferred_element_type=jnp.float32,
                         precision=lax.Precision.HIGHEST) + b3_ref[...]
        m = jnp.max(logits, axis=-1, keepdims=True)
        lse = m + jnp.log(jnp.sum(jnp.exp(logits - m), axis=-1, keepdims=True))
        o_ref[...] = logits - lse

    return pl.pallas_call(
        body,
        out_shape=jax.ShapeDtypeStruct((N, O), jnp.float32),
    )(accp, y2, dis, w3, b2, b3)


def kernel(x, edge_index, W1, b1, W2, b2, W3, b3):
    src3 = edge_index[0].reshape(NW, KJ, CB)
    dst3 = edge_index[1].reshape(NW, KJ, CB)
    ones_u = jnp.ones((CB, D), jnp.float32)
    zeros_nd = jnp.zeros((N, D), jnp.float32)

    deg_parts = _sc_degree(dst3, ones_u, zeros_nd)
    dis, y1 = _tc_stage1(deg_parts, x, W1)
    acc1 = _sc_scatter_rows(y1, src3, dst3, zeros_nd)
    y2 = _tc_stage2(acc1, y1, dis, W2, b1)
    acc2 = _sc_scatter_rows(y2, src3, dst3, zeros_nd)
    return _tc_stage3(acc2, y2, dis, W3, b2, b3)


# speedup vs baseline: 19.2739x; 1.0233x over previous
import functools

import jax
import jax.numpy as jnp
from jax import lax
from jax.experimental import pallas as pl
from jax.experimental.pallas import tpu as pltpu
from jax.experimental.pallas import tpu_sc as plsc

N = 10000
E = 320000
D = 128
O = 2

NC = 2
NS = 16
NW = NC * NS
CB = 50
KJ = E // (NW * CB)
ZS = N // 10


NB = 2
PHS = ((0, 104), (104, 96))
PHR = max(n for _, n in PHS)
FK = 4


def _sc_mesh():
    return plsc.VectorSubcoreMesh(core_axis_name="c", subcore_axis_name="s")


def _fill(ref2d, val):
    v = jnp.full((16,), val, jnp.float32)

    def body(i, carry):
        for l in range(D // 16):
            ref2d[i, pl.ds(l * 16, 16)] = v
        return carry

    lax.fori_loop(0, CB, body, 0)


def _zero_stripe(zsrc, acc_sh, s):
    @pl.when(s < 10)
    def _():
        def body(t, carry):
            pltpu.sync_copy(zsrc, acc_sh.at[pl.ds(s * ZS + t * CB, CB)])
            return carry

        lax.fori_loop(0, ZS // CB, body, 0)


def _write_stripe(acc_sh, out_hbm, c, s):
    @pl.when(s < 10)
    def _():
        pltpu.sync_copy(acc_sh.at[pl.ds(s * ZS, ZS)],
                        out_hbm.at[c, pl.ds(s * ZS, ZS)])


def _sc_degree(dst3):
    @functools.partial(
        pl.kernel,
        out_type=jax.ShapeDtypeStruct((NC, N, D), jnp.float32),
        mesh=_sc_mesh(),
        scratch_types=[
            pltpu.VMEM((KJ, CB), jnp.int32),
            pltpu.VMEM((CB, D), jnp.float32),
            pltpu.VMEM_SHARED((N, D), jnp.float32),
            pltpu.SemaphoreType.DMA,
        ],
    )
    def k(dst_hbm, out_hbm, dst_v, ones_v, deg_sh, sem):
        c = lax.axis_index("c")
        s = lax.axis_index("s")
        wid = s * NC + c

        pltpu.sync_copy(dst_hbm.at[wid], dst_v)
        _fill(ones_v, 0.0)
        _zero_stripe(ones_v, deg_sh, s)
        _fill(ones_v, 1.0)
        plsc.subcore_barrier()

        def outer(g, carry):
            handles = [
                pltpu.async_copy(ones_v, deg_sh.at[dst_v.at[g * FK + b]],
                                 sem, add=True)
                for b in range(FK)
            ]
            for h in handles:
                h.wait()
            return carry

        lax.fori_loop(0, KJ // FK, outer, 0)
        plsc.subcore_barrier()
        _write_stripe(deg_sh, out_hbm, c, s)

    return k(dst3)


def _sc_scatter_rows(y, src3, dst3):
    @functools.partial(
        pl.kernel,
        out_type=jax.ShapeDtypeStruct((NC, N, D), jnp.float32),
        mesh=_sc_mesh(),
        scratch_types=[
            pltpu.VMEM((PHR, CB), jnp.int32),
            pltpu.VMEM((PHR, CB), jnp.int32),
            pltpu.VMEM((NB, CB, D), jnp.float32),
            pltpu.VMEM_SHARED((N, D), jnp.float32),
        ] + [pltpu.SemaphoreType.DMA] * NB,
    )
    def k(y_hbm, src_hbm, dst_hbm, out_hbm, src_v, dst_v, rows_v, acc_sh,
          *gsem):
        c = lax.axis_index("c")
        s = lax.axis_index("s")
        wid = s * NC + c

        _fill(rows_v.at[0], 0.0)
        _zero_stripe(rows_v.at[0], acc_sh, s)
        plsc.subcore_barrier()

        for off, nblk in PHS:
            pltpu.sync_copy(src_hbm.at[wid, pl.ds(off, nblk)],
                            src_v.at[pl.ds(0, nblk)])
            pltpu.sync_copy(dst_hbm.at[wid, pl.ds(off, nblk)],
                            dst_v.at[pl.ds(0, nblk)])

            def outer(g, carry):
                handles = [
                    pltpu.async_copy(y_hbm.at[src_v.at[g * NB + b]],
                                     rows_v.at[b], gsem[b])
                    for b in range(NB)
                ]
                for b in range(NB):
                    handles[b].wait()
                    pltpu.sync_copy(rows_v.at[b],
                                    acc_sh.at[dst_v.at[g * NB + b]],
                                    add=True)
                return carry

            lax.fori_loop(0, nblk // NB, outer, 0)

        plsc.subcore_barrier()
        _write_stripe(acc_sh, out_hbm, c, s)

    return k(y, src3, dst3)


def _tc_stage1(deg_t, x, w1):
    def body(deg_ref, x_ref, w_ref, dis_ref, y_ref):
        d0 = deg_ref[0]
        d1 = deg_ref[1]
        deg = d0[:, 0:1] + d1[:, 0:1] + 1.0
        dis = lax.rsqrt(deg)
        dis_ref[...] = dis
        xw = jnp.dot(x_ref[...], w_ref[...],
                     preferred_element_type=jnp.float32,
                     precision=lax.Precision.HIGHEST)
        y_ref[...] = xw * dis

    return pl.pallas_call(
        body,
        out_shape=[jax.ShapeDtypeStruct((N, 1), jnp.float32),
                   jax.ShapeDtypeStruct((N, D), jnp.float32)],
    )(deg_t, x, w1)


def _tc_stage2(accp, y1, dis, w2, b1):
    def body(a_ref, y_ref, d_ref, w_ref, b_ref, y2_ref):
        acc = a_ref[0] + a_ref[1] + y_ref[...]
        h = jnp.maximum(d_ref[...] * acc + b_ref[...], 0.0)
        hw = jnp.dot(h, w_ref[...],
                     preferred_element_type=jnp.float32,
                     precision=lax.Precision.HIGHEST)
        y2_ref[...] = hw * d_ref[...]

    return pl.pallas_call(
        body,
        out_shape=jax.ShapeDtypeStruct((N, D), jnp.float32),
    )(accp, y1, dis, w2, b1)


def _tc_stage3(accp, y2, dis, w3, b2, b3):
    def body(a_ref, y_ref, d_ref, w_ref, b2_ref, b3_ref, o_ref):
        acc = a_ref[0] + a_ref[1] + y_ref[...]
        h = jnp.maximum(d_ref[...] * acc + b2_ref[...], 0.0)
        logits = jnp.dot(h, w_ref[...],
                         preferred_element_type=jnp.float32,
                         precision=lax.Precision.HIGHEST) + b3_ref[...]
        m = jnp.max(logits, axis=-1, keepdims=True)
        lse = m + jnp.log(jnp.sum(jnp.exp(logits - m), axis=-1, keepdims=True))
        o_ref[...] = logits - lse

    return pl.pallas_call(
        body,
        out_shape=jax.ShapeDtypeStruct((N, O), jnp.float32),
    )(accp, y2, dis, w3, b2, b3)


def kernel(x, edge_index, W1, b1, W2, b2, W3, b3):
    src3 = edge_index[0].reshape(NW, KJ, CB)
    dst3 = edge_index[1].reshape(NW, KJ, CB)
    deg_parts = _sc_degree(dst3)
    dis, y1 = _tc_stage1(deg_parts, x, W1)
    acc1 = _sc_scatter_rows(y1, src3, dst3)
    y2 = _tc_stage2(acc1, y1, dis, W2, b1)
    acc2 = _sc_scatter_rows(y2, src3, dst3)
    return _tc_stage3(acc2, y2, dis, W3, b2, b3)
